# baseline (speedup 1.0000x reference)
import functools

import jax
import jax.numpy as jnp
from jax import lax
from jax.experimental import pallas as pl
from jax.experimental.pallas import tpu as pltpu
from jax.experimental.pallas import tpu_sc as plsc

DIM = 64
BATCH = 16384
NC = 2
NS = 16
NW = NC * NS
BPW = BATCH // NW


def _sc_gather(user_table, item_table, user, pos, neg):
    mesh = plsc.VectorSubcoreMesh(core_axis_name="c", subcore_axis_name="s")
    out_t = jax.ShapeDtypeStruct((BATCH, DIM), jnp.float32)

    @functools.partial(
        pl.kernel,
        mesh=mesh,
        out_type=[out_t, out_t, out_t],
        compiler_params=pltpu.CompilerParams(use_tc_tiling_on_sc=False),
        scratch_types=[
            pltpu.VMEM((BPW,), jnp.int32),
            pltpu.VMEM((BPW,), jnp.int32),
            pltpu.VMEM((BPW,), jnp.int32),
            pltpu.VMEM((BPW, DIM), jnp.float32),
            pltpu.VMEM((BPW, DIM), jnp.float32),
            pltpu.VMEM((BPW, DIM), jnp.float32),
            pltpu.SemaphoreType.DMA,
            pltpu.SemaphoreType.DMA,
            pltpu.SemaphoreType.DMA,
        ],
    )
    def k(ut_hbm, it_hbm, u_hbm, p_hbm, n_hbm, ue_hbm, pe_hbm, ne_hbm,
          ui_v, pi_v, ni_v, ur_v, pr_v, nr_v, su, sp, sn):
        wid = lax.axis_index("s") * NC + lax.axis_index("c")
        base = wid * BPW
        pltpu.sync_copy(u_hbm.at[pl.ds(base, BPW)], ui_v)
        pltpu.sync_copy(p_hbm.at[pl.ds(base, BPW)], pi_v)
        pltpu.sync_copy(n_hbm.at[pl.ds(base, BPW)], ni_v)
        cu = pltpu.async_copy(ut_hbm.at[ui_v], ur_v, su)
        cp = pltpu.async_copy(it_hbm.at[pi_v], pr_v, sp)
        cn = pltpu.async_copy(it_hbm.at[ni_v], nr_v, sn)
        cu.wait()
        pltpu.sync_copy(ur_v, ue_hbm.at[pl.ds(base, BPW)])
        cp.wait()
        pltpu.sync_copy(pr_v, pe_hbm.at[pl.ds(base, BPW)])
        cn.wait()
        pltpu.sync_copy(nr_v, ne_hbm.at[pl.ds(base, BPW)])

    return k(user_table, item_table, user, pos, neg)


_TC_BLK = 2048


def _tc_loss_body(u_ref, p_ref, n_ref, o_ref):
    t = jnp.sum(u_ref[...] * (p_ref[...] - n_ref[...]), axis=1)
    part = jnp.sum(jnp.logaddexp(0.0, -t))

    @pl.when(pl.program_id(0) == 0)
    def _():
        o_ref[0] = 0.0

    o_ref[0] += part


def _tc_loss(ue, pe, ne):
    spec = pl.BlockSpec((_TC_BLK, DIM), lambda i: (i, 0))
    out = pl.pallas_call(
        _tc_loss_body,
        grid=(BATCH // _TC_BLK,),
        in_specs=[spec, spec, spec],
        out_specs=pl.BlockSpec(memory_space=pltpu.SMEM),
        out_shape=jax.ShapeDtypeStruct((1,), jnp.float32),
    )(ue, pe, ne)
    return out[0]


def kernel(user_table, item_table, user, pos, neg):
    ue, pe, ne = _sc_gather(user_table, item_table, user, pos, neg)
    return _tc_loss(ue, pe, ne)
